# Initial kernel scaffold; baseline (speedup 1.0000x reference)
import jax
import jax.numpy as jnp
from jax.experimental import pallas as pl


def kernel(user_gmf_table, item_gmf_table, user_mlp_table, item_mlp_table, W0, b0, W1, b1, W2, b2, Wo, bo, user_indices, item_indices):
    raise NotImplementedError("write your pallas kernel here")


# speedup vs baseline: 2.5190x; 2.5190x over previous
import functools

import jax
import jax.numpy as jnp
from jax import lax
from jax.experimental import pallas as pl
from jax.experimental.pallas import tpu as pltpu
from jax.experimental.pallas import tpu_sc as plsc

B = 16384
EMB = 128
NC = 2
NS = 16
NW = NC * NS
BPW = B // NW
CHUNK = 128
NCHUNK = BPW // CHUNK


def _sc_gather(ug, ig, um, im, uidx, iidx):
    mesh = plsc.VectorSubcoreMesh(core_axis_name="c", subcore_axis_name="s")

    @functools.partial(
        pl.kernel,
        mesh=mesh,
        out_type=[jax.ShapeDtypeStruct((B, EMB), jnp.float32)] * 4,
        scratch_types=[
            pltpu.VMEM((NCHUNK, CHUNK), jnp.int32),
            pltpu.VMEM((NCHUNK, CHUNK), jnp.int32),
            pltpu.VMEM((BPW, EMB), jnp.float32),
            pltpu.SemaphoreType.DMA,
        ],
    )
    def k(ug_h, ig_h, um_h, im_h, uidx_h, iidx_h,
          o_ug, o_ig, o_um, o_im, uv, iv, rows, sem):
        wid = lax.axis_index("s") * NC + lax.axis_index("c")
        base = wid * BPW
        pltpu.sync_copy(uidx_h.at[wid], uv)
        pltpu.sync_copy(iidx_h.at[wid], iv)
        for table, idxv, out in ((ug_h, uv, o_ug), (ig_h, iv, o_ig),
                                 (um_h, uv, o_um), (im_h, iv, o_im)):
            copies = [
                pltpu.async_copy(table.at[idxv.at[j]],
                                 rows.at[pl.ds(j * CHUNK, CHUNK)], sem)
                for j in range(NCHUNK)
            ]
            for c in copies:
                c.wait()
            pltpu.sync_copy(rows, out.at[pl.ds(base, BPW)])

    return k(ug, ig, um, im, uidx, iidx)


def _tc_body(gu_r, gi_r, mu_r, mi_r, w0a_r, w0b_r, b0_r, w1_r, b1_r,
             w2_r, b2_r, wog_r, wox_r, bo_r, o_r):
    x = jnp.maximum(mu_r[...] @ w0a_r[...] + mi_r[...] @ w0b_r[...] + b0_r[...], 0.0)
    x = jnp.maximum(x @ w1_r[...] + b1_r[...], 0.0)
    x = jnp.maximum(x @ w2_r[...] + b2_r[...], 0.0)
    g = gu_r[...] * gi_r[...]
    p = g @ wog_r[...] + x @ wox_r[...] + bo_r[...]
    o_r[...] = jax.nn.sigmoid(p)


def _tc_forward(gu, gi, mu, mi, W0, b0, W1, b1, W2, b2, Wo, bo, bb=2048):
    w0a, w0b = W0[:EMB], W0[EMB:]
    wog, wox = Wo[:EMB], Wo[EMB:]
    full = lambda i: (0, 0)
    out = pl.pallas_call(
        _tc_body,
        grid=(B // bb,),
        in_specs=[
            pl.BlockSpec((bb, EMB), lambda i: (i, 0)),
            pl.BlockSpec((bb, EMB), lambda i: (i, 0)),
            pl.BlockSpec((bb, EMB), lambda i: (i, 0)),
            pl.BlockSpec((bb, EMB), lambda i: (i, 0)),
            pl.BlockSpec((EMB, 256), full),
            pl.BlockSpec((EMB, 256), full),
            pl.BlockSpec((1, 256), full),
            pl.BlockSpec((256, 128), full),
            pl.BlockSpec((1, 128), full),
            pl.BlockSpec((128, 64), full),
            pl.BlockSpec((1, 64), full),
            pl.BlockSpec((EMB, 1), full),
            pl.BlockSpec((64, 1), full),
            pl.BlockSpec((1, 1), full),
        ],
        out_specs=pl.BlockSpec((bb, 1), lambda i: (i, 0)),
        out_shape=jax.ShapeDtypeStruct((B, 1), jnp.float32),
    )(gu, gi, mu, mi, w0a, w0b, b0.reshape(1, 256), W1, b1.reshape(1, 128),
      W2, b2.reshape(1, 64), wog, wox, bo.reshape(1, 1))
    return out


def kernel(user_gmf_table, item_gmf_table, user_mlp_table, item_mlp_table,
           W0, b0, W1, b1, W2, b2, Wo, bo, user_indices, item_indices):
    uidx = user_indices.astype(jnp.int32).reshape(NW, NCHUNK, CHUNK)
    iidx = item_indices.astype(jnp.int32).reshape(NW, NCHUNK, CHUNK)
    gu, gi, mu, mi = _sc_gather(user_gmf_table, item_gmf_table,
                                user_mlp_table, item_mlp_table, uidx, iidx)
    out = _tc_forward(gu, gi, mu, mi, W0, b0, W1, b1, W2, b2, Wo, bo)
    return out.squeeze(-1)
